# baseline (speedup 1.0000x reference)
import functools

import jax
import jax.numpy as jnp
from jax import lax
from jax.experimental import pallas as pl
from jax.experimental.pallas import tpu as pltpu
from jax.experimental.pallas import tpu_sc as plsc

N = 10000
E = 320000
DF = 128
DIM = 32
G = 64

NC = 2
NS = 16
NW = NC * NS
NPAD = 10240
EPAD = 327680
EPW = EPAD // NW
CW = 128
CHUNKS = EPW // CW
KG = 8
CHUNKS_A = CHUNKS + 2 * KG
STRIPE = NPAD // NS

RB = 1280
GRID = NPAD // RB

DEGW = DIM


def _deg_sc_body(dst_hbm, out_hbm, dst_v, ones_v, stage_v, acc_sh, sem):
    c = lax.axis_index("c")
    s = lax.axis_index("s")
    wid = s * NC + c
    pltpu.sync_copy(dst_hbm.at[wid], dst_v)

    one16 = jnp.ones((16,), jnp.float32)
    zero16 = jnp.zeros((16,), jnp.float32)

    def fill_body(i, carry):
        ones_v[i, pl.ds(0, 16)] = one16
        ones_v[i, pl.ds(16, 16)] = one16
        return carry

    lax.fori_loop(0, CW, fill_body, 0, unroll=8)

    def zero_body(i, carry):
        stage_v[i, pl.ds(0, 16)] = zero16
        stage_v[i, pl.ds(16, 16)] = zero16
        return carry

    lax.fori_loop(0, STRIPE, zero_body, 0, unroll=8)
    pltpu.sync_copy(stage_v, acc_sh.at[pl.ds(s * STRIPE, STRIPE)])
    plsc.subcore_barrier()

    def body(j, carry):
        pltpu.async_copy(ones_v, acc_sh.at[dst_v.at[j]], sem, add=True)
        return carry

    lax.fori_loop(0, CHUNKS, body, 0)

    def drain(j, carry):
        pltpu.make_async_copy(ones_v, acc_sh.at[dst_v.at[0]], sem).wait()
        return carry

    lax.fori_loop(0, CHUNKS, drain, 0)
    plsc.subcore_barrier()

    pltpu.sync_copy(acc_sh.at[pl.ds(s * STRIPE, STRIPE)], stage_v)
    pltpu.sync_copy(stage_v, out_hbm.at[c, pl.ds(s * STRIPE, STRIPE)])


def _edge_sc_body(g_hbm, src_hbm, dst_hbm, out_hbm,
                  src_v, dst_v, rows_a, rows_b, stage_v, acc_sh,
                  gsem_a, gsem_b, ssem_a, ssem_b):
    c = lax.axis_index("c")
    s = lax.axis_index("s")
    wid = s * NC + c
    pltpu.sync_copy(src_hbm.at[wid], src_v)
    pltpu.sync_copy(dst_hbm.at[wid], dst_v)

    zero16 = jnp.zeros((16,), jnp.float32)

    def zero_body(i, carry):
        stage_v[i, pl.ds(0, 16)] = zero16
        stage_v[i, pl.ds(16, 16)] = zero16
        return carry

    lax.fori_loop(0, STRIPE, zero_body, 0, unroll=8)
    pltpu.sync_copy(stage_v, acc_sh.at[pl.ds(s * STRIPE, STRIPE)])
    plsc.subcore_barrier()

    def fire_gathers(grp, buf, gsem):
        for b in range(KG):
            pltpu.async_copy(g_hbm.at[src_v.at[grp * KG + b]], buf.at[b], gsem)

    def wait_gathers(buf, gsem):
        for b in range(KG):
            pltpu.make_async_copy(g_hbm.at[src_v.at[0]], buf.at[b], gsem).wait()

    def fire_scatters(grp, buf, ssem):
        for b in range(KG):
            pltpu.async_copy(buf.at[b], acc_sh.at[dst_v.at[grp * KG + b]],
                             ssem, add=True)

    def wait_scatters(buf, ssem):
        for b in range(KG):
            pltpu.make_async_copy(buf.at[b], acc_sh.at[dst_v.at[0]],
                                  ssem).wait()

    fire_gathers(0, rows_a, gsem_a)
    fire_gathers(1, rows_b, gsem_b)

    def body(i, carry):
        ga = 2 * i
        wait_gathers(rows_a, gsem_a)
        fire_scatters(ga, rows_a, ssem_a)
        wait_gathers(rows_b, gsem_b)
        fire_scatters(ga + 1, rows_b, ssem_b)
        wait_scatters(rows_a, ssem_a)
        fire_gathers(ga + 2, rows_a, gsem_a)
        wait_scatters(rows_b, ssem_b)
        fire_gathers(ga + 3, rows_b, gsem_b)
        return carry

    lax.fori_loop(0, CHUNKS // (2 * KG), body, 0)
    wait_gathers(rows_a, gsem_a)
    wait_gathers(rows_b, gsem_b)
    plsc.subcore_barrier()

    pltpu.sync_copy(acc_sh.at[pl.ds(s * STRIPE, STRIPE)], stage_v)
    pltpu.sync_copy(stage_v, out_hbm.at[c, pl.ds(s * STRIPE, STRIPE)])


@functools.lru_cache(maxsize=1)
def _sc_kernels():
    mesh = plsc.VectorSubcoreMesh(
        core_axis_name="c", subcore_axis_name="s",
        num_cores=NC, num_subcores=NS)
    deg = pl.kernel(
        _deg_sc_body,
        out_type=jax.ShapeDtypeStruct((NC, NPAD, DEGW), jnp.float32),
        mesh=mesh,
        compiler_params=pltpu.CompilerParams(use_tc_tiling_on_sc=False),
        scratch_types=[
            pltpu.VMEM((CHUNKS, CW), jnp.int32),
            pltpu.VMEM((CW, DEGW), jnp.float32),
            pltpu.VMEM((STRIPE, DEGW), jnp.float32),
            pltpu.VMEM_SHARED((NPAD, DEGW), jnp.float32),
            pltpu.SemaphoreType.DMA,
        ],
    )
    edge = pl.kernel(
        _edge_sc_body,
        out_type=jax.ShapeDtypeStruct((NC, NPAD, DIM), jnp.float32),
        mesh=mesh,
        compiler_params=pltpu.CompilerParams(use_tc_tiling_on_sc=False),
        scratch_types=[
            pltpu.VMEM((CHUNKS_A, CW), jnp.int32),
            pltpu.VMEM((CHUNKS_A, CW), jnp.int32),
            pltpu.VMEM((KG, CW, DIM), jnp.float32),
            pltpu.VMEM((KG, CW, DIM), jnp.float32),
            pltpu.VMEM((STRIPE, DIM), jnp.float32),
            pltpu.VMEM_SHARED((NPAD, DIM), jnp.float32),
            pltpu.SemaphoreType.DMA,
            pltpu.SemaphoreType.DMA,
            pltpu.SemaphoreType.DMA,
            pltpu.SemaphoreType.DMA,
        ],
    )
    return deg, edge


NPAD4 = NPAD // 4
N4 = N // 4
RB4 = 640
GRID4 = NPAD4 // RB4


def _pre_body(x_ref, w_ref, o_ref):
    o_ref[...] = jnp.dot(x_ref[...], w_ref[...],
                         preferred_element_type=jnp.float32)


_pre_tc = pl.pallas_call(
    _pre_body,
    grid=(GRID4,),
    in_specs=[
        pl.BlockSpec((RB4, 4 * DF), lambda i: (i, 0)),
        pl.BlockSpec((4 * DF, 128), lambda i: (0, 0)),
    ],
    out_specs=pl.BlockSpec((RB4, 128), lambda i: (i, 0)),
    out_shape=jax.ShapeDtypeStruct((NPAD4, 128), jnp.float32),
)


def _dinv_g1_body(dp_ref, p_ref, dinv_ref, g_ref):
    dv = lax.rsqrt(dp_ref[0] + dp_ref[1] + 2.0)
    dinv_ref[...] = dv
    g_ref[...] = dv * p_ref[...]


_dinv_g1_tc = pl.pallas_call(
    _dinv_g1_body,
    grid=(GRID4,),
    in_specs=[
        pl.BlockSpec((2, RB4, 128), lambda i: (0, i, 0)),
        pl.BlockSpec((RB4, 128), lambda i: (i, 0)),
    ],
    out_specs=[
        pl.BlockSpec((RB4, 128), lambda i: (i, 0)),
        pl.BlockSpec((RB4, 128), lambda i: (i, 0)),
    ],
    out_shape=[
        jax.ShapeDtypeStruct((NPAD4, 128), jnp.float32),
        jax.ShapeDtypeStruct((NPAD4, 128), jnp.float32),
    ],
)


def _mid_body(s_ref, g_ref, dinv_ref, w_ref, b_ref, o_ref):
    i = pl.program_id(0)
    dv = dinv_ref[...]
    h = jnp.maximum(dv * (s_ref[0] + s_ref[1] + 2.0 * g_ref[...])
                    + b_ref[...], 0.0)
    p = jnp.dot(h, w_ref[...], preferred_element_type=jnp.float32)
    rows = i * RB4 + lax.broadcasted_iota(jnp.int32, (RB4, 1), 0)
    o_ref[...] = jnp.where(rows < N4, dv * p, 0.0)


_mid_tc = pl.pallas_call(
    _mid_body,
    grid=(GRID4,),
    in_specs=[
        pl.BlockSpec((2, RB4, 128), lambda i: (0, i, 0)),
        pl.BlockSpec((RB4, 128), lambda i: (i, 0)),
        pl.BlockSpec((RB4, 128), lambda i: (i, 0)),
        pl.BlockSpec((128, 128), lambda i: (0, 0)),
        pl.BlockSpec((1, 128), lambda i: (0, 0)),
    ],
    out_specs=pl.BlockSpec((RB4, 128), lambda i: (i, 0)),
    out_shape=jax.ShapeDtypeStruct((NPAD4, 128), jnp.float32),
)


def _final_body(s_ref, g_ref, dinv_ref, b_ref, bt_ref, wo_ref, bo_ref,
                o_ref, acc_ref):
    i = pl.program_id(0)

    @pl.when(i == 0)
    def _init():
        acc_ref[...] = jnp.zeros_like(acc_ref)

    h = jnp.maximum(
        dinv_ref[...] * (s_ref[0] + s_ref[1] + 2.0 * g_ref[...]) + b_ref[...],
        0.0)
    for q in range(4):
        bq = bt_ref[q:q + 1, :]
        oh = (lax.broadcasted_iota(jnp.int32, (G, RB4), 0)
              == jnp.broadcast_to(bq, (G, RB4))).astype(jnp.float32)
        acc_ref[...] += jnp.dot(oh, h[:, 32 * q:32 * (q + 1)],
                                preferred_element_type=jnp.float32)

    @pl.when(i == pl.num_programs(0) - 1)
    def _emit():
        o_ref[...] = (jnp.dot(acc_ref[...], wo_ref[...],
                              preferred_element_type=jnp.float32)
                      + bo_ref[...])


_final_tc = pl.pallas_call(
    _final_body,
    grid=(GRID4,),
    in_specs=[
        pl.BlockSpec((2, RB4, 128), lambda i: (0, i, 0)),
        pl.BlockSpec((RB4, 128), lambda i: (i, 0)),
        pl.BlockSpec((RB4, 128), lambda i: (i, 0)),
        pl.BlockSpec((1, 128), lambda i: (0, 0)),
        pl.BlockSpec((4, RB4), lambda i: (0, i)),
        pl.BlockSpec((DIM, 1), lambda i: (0, 0)),
        pl.BlockSpec((1, 1), lambda i: (0, 0)),
    ],
    out_specs=pl.BlockSpec((G, 1), lambda i: (0, 0)),
    out_shape=jax.ShapeDtypeStruct((G, 1), jnp.float32),
    scratch_shapes=[pltpu.VMEM((G, DIM), jnp.float32)],
)


def kernel(x, edge_index, batch, W1, b1, W2, b2, Wo, bo):
    src = edge_index[0]
    dst = edge_index[1]
    npad_rows = NPAD - N
    pad_idx = N + (jnp.arange(EPAD - E, dtype=jnp.int32) % npad_rows)
    srcp = jnp.concatenate([src, pad_idx]).reshape(NW, CHUNKS, CW)
    dstp = jnp.concatenate([dst, pad_idx]).reshape(NW, CHUNKS, CW)
    extra = N + (jnp.arange(NW * 2 * KG * CW, dtype=jnp.int32)
                 % npad_rows).reshape(NW, 2 * KG, CW)
    srcpa = jnp.concatenate([srcp, extra], axis=1)
    dstpa = jnp.concatenate([dstp, extra], axis=1)
    x4 = jnp.pad(x, ((0, npad_rows), (0, 0))).reshape(NPAD4, 4 * DF)
    batch4 = jnp.concatenate(
        [batch, jnp.full((npad_rows,), G, jnp.int32)]).reshape(NPAD4, 4).T
    eye4 = jnp.eye(4, dtype=jnp.float32)
    w1bd = jnp.kron(eye4, W1)
    w2bd = jnp.kron(eye4, W2)
    b1t = jnp.tile(b1, 4).reshape(1, 128)
    b2t = jnp.tile(b2, 4).reshape(1, 128)
    bor = bo.reshape(1, 1)

    _deg_sc, _edge_sc = _sc_kernels()
    deg_parts = _deg_sc(dstp)
    degs = deg_parts.reshape(NC, NPAD4, 128)
    p1 = _pre_tc(x4, w1bd)
    dinvp, g1p = _dinv_g1_tc(degs, p1)

    s1 = _edge_sc(g1p.reshape(NPAD, DIM), srcpa, dstpa)
    g2p = _mid_tc(s1.reshape(NC, NPAD4, 128), g1p, dinvp, w2bd, b1t)
    s2 = _edge_sc(g2p.reshape(NPAD, DIM), srcpa, dstpa)
    g3p = _mid_tc(s2.reshape(NC, NPAD4, 128), g2p, dinvp, w2bd, b2t)
    s3 = _edge_sc(g3p.reshape(NPAD, DIM), srcpa, dstpa)
    return _final_tc(s3.reshape(NC, NPAD4, 128), g3p, dinvp, b2t, batch4,
                     Wo, bor)


# speedup vs baseline: 1.1491x; 1.1491x over previous
import functools

import jax
import jax.numpy as jnp
from jax import lax
from jax.experimental import pallas as pl
from jax.experimental.pallas import tpu as pltpu
from jax.experimental.pallas import tpu_sc as plsc

N = 10000
E = 320000
DF = 128
DIM = 32
G = 64

NC = 2
NS = 16
NW = NC * NS
NPAD = 10240
EPAD = 327680
EPW = EPAD // NW
CW = 128
CHUNKS = EPW // CW
KG = 2
CHUNKS_A = CHUNKS + 2 * KG
STRIPE = NPAD // NS

RB = 1280
GRID = NPAD // RB

DEGW = DIM


def _deg_sc_body(dst_hbm, out_hbm, dst_v, ones_v, stage_v, acc_sh, sem):
    c = lax.axis_index("c")
    s = lax.axis_index("s")
    wid = s * NC + c
    pltpu.sync_copy(dst_hbm.at[wid], dst_v)

    one16 = jnp.ones((16,), jnp.float32)
    zero16 = jnp.zeros((16,), jnp.float32)

    def fill_body(i, carry):
        ones_v[i, pl.ds(0, 16)] = one16
        ones_v[i, pl.ds(16, 16)] = one16
        return carry

    lax.fori_loop(0, CW, fill_body, 0, unroll=8)

    def zero_body(i, carry):
        stage_v[i, pl.ds(0, 16)] = zero16
        stage_v[i, pl.ds(16, 16)] = zero16
        return carry

    lax.fori_loop(0, STRIPE, zero_body, 0, unroll=8)
    pltpu.sync_copy(stage_v, acc_sh.at[pl.ds(s * STRIPE, STRIPE)])
    plsc.subcore_barrier()

    def body(j, carry):
        pltpu.async_copy(ones_v, acc_sh.at[dst_v.at[j]], sem, add=True)
        return carry

    lax.fori_loop(0, CHUNKS, body, 0)

    def drain(j, carry):
        pltpu.make_async_copy(ones_v, acc_sh.at[dst_v.at[0]], sem).wait()
        return carry

    lax.fori_loop(0, CHUNKS, drain, 0)
    plsc.subcore_barrier()

    pltpu.sync_copy(acc_sh.at[pl.ds(s * STRIPE, STRIPE)], stage_v)
    pltpu.sync_copy(stage_v, out_hbm.at[c, pl.ds(s * STRIPE, STRIPE)])


def _edge_sc_body(g_hbm, src_hbm, dst_hbm, out_hbm,
                  src_v, dst_v, rows_a, rows_b, stage_v, acc_sh,
                  gsem_a, gsem_b, ssem_a, ssem_b):
    c = lax.axis_index("c")
    s = lax.axis_index("s")
    wid = s * NC + c
    pltpu.sync_copy(src_hbm.at[wid], src_v)
    pltpu.sync_copy(dst_hbm.at[wid], dst_v)

    zero16 = jnp.zeros((16,), jnp.float32)

    def zero_body(i, carry):
        stage_v[i, pl.ds(0, 16)] = zero16
        stage_v[i, pl.ds(16, 16)] = zero16
        return carry

    lax.fori_loop(0, STRIPE, zero_body, 0, unroll=8)
    pltpu.sync_copy(stage_v, acc_sh.at[pl.ds(s * STRIPE, STRIPE)])
    plsc.subcore_barrier()

    def fire_gathers(grp, buf, gsem):
        for b in range(KG):
            pltpu.async_copy(g_hbm.at[src_v.at[grp * KG + b]], buf.at[b], gsem)

    def wait_gathers(buf, gsem):
        for b in range(KG):
            pltpu.make_async_copy(g_hbm.at[src_v.at[0]], buf.at[b], gsem).wait()

    def fire_scatters(grp, buf, ssem):
        for b in range(KG):
            pltpu.async_copy(buf.at[b], acc_sh.at[dst_v.at[grp * KG + b]],
                             ssem, add=True)

    def wait_scatters(buf, ssem):
        for b in range(KG):
            pltpu.make_async_copy(buf.at[b], acc_sh.at[dst_v.at[0]],
                                  ssem).wait()

    fire_gathers(0, rows_a, gsem_a)
    fire_gathers(1, rows_b, gsem_b)

    def body(i, carry):
        ga = 2 * i
        wait_gathers(rows_a, gsem_a)
        fire_scatters(ga, rows_a, ssem_a)
        wait_gathers(rows_b, gsem_b)
        fire_scatters(ga + 1, rows_b, ssem_b)
        wait_scatters(rows_a, ssem_a)
        fire_gathers(ga + 2, rows_a, gsem_a)
        wait_scatters(rows_b, ssem_b)
        fire_gathers(ga + 3, rows_b, gsem_b)
        return carry

    lax.fori_loop(0, CHUNKS // (2 * KG), body, 0)
    wait_gathers(rows_a, gsem_a)
    wait_gathers(rows_b, gsem_b)
    plsc.subcore_barrier()

    pltpu.sync_copy(acc_sh.at[pl.ds(s * STRIPE, STRIPE)], stage_v)
    pltpu.sync_copy(stage_v, out_hbm.at[c, pl.ds(s * STRIPE, STRIPE)])


@functools.lru_cache(maxsize=1)
def _sc_kernels():
    mesh = plsc.VectorSubcoreMesh(
        core_axis_name="c", subcore_axis_name="s",
        num_cores=NC, num_subcores=NS)
    deg = pl.kernel(
        _deg_sc_body,
        out_type=jax.ShapeDtypeStruct((NC, NPAD, DEGW), jnp.float32),
        mesh=mesh,
        compiler_params=pltpu.CompilerParams(use_tc_tiling_on_sc=False),
        scratch_types=[
            pltpu.VMEM((CHUNKS, CW), jnp.int32),
            pltpu.VMEM((CW, DEGW), jnp.float32),
            pltpu.VMEM((STRIPE, DEGW), jnp.float32),
            pltpu.VMEM_SHARED((NPAD, DEGW), jnp.float32),
            pltpu.SemaphoreType.DMA,
        ],
    )
    edge = pl.kernel(
        _edge_sc_body,
        out_type=jax.ShapeDtypeStruct((NC, NPAD, DIM), jnp.float32),
        mesh=mesh,
        compiler_params=pltpu.CompilerParams(use_tc_tiling_on_sc=False),
        scratch_types=[
            pltpu.VMEM((CHUNKS_A, CW), jnp.int32),
            pltpu.VMEM((CHUNKS_A, CW), jnp.int32),
            pltpu.VMEM((KG, CW, DIM), jnp.float32),
            pltpu.VMEM((KG, CW, DIM), jnp.float32),
            pltpu.VMEM((STRIPE, DIM), jnp.float32),
            pltpu.VMEM_SHARED((NPAD, DIM), jnp.float32),
            pltpu.SemaphoreType.DMA,
            pltpu.SemaphoreType.DMA,
            pltpu.SemaphoreType.DMA,
            pltpu.SemaphoreType.DMA,
        ],
    )
    return deg, edge


NPAD4 = NPAD // 4
N4 = N // 4
RB4 = 640
GRID4 = NPAD4 // RB4


def _pre_body(x_ref, w_ref, o_ref):
    o_ref[...] = jnp.dot(x_ref[...], w_ref[...],
                         preferred_element_type=jnp.float32)


_pre_tc = pl.pallas_call(
    _pre_body,
    grid=(GRID4,),
    in_specs=[
        pl.BlockSpec((RB4, 4 * DF), lambda i: (i, 0)),
        pl.BlockSpec((4 * DF, 128), lambda i: (0, 0)),
    ],
    out_specs=pl.BlockSpec((RB4, 128), lambda i: (i, 0)),
    out_shape=jax.ShapeDtypeStruct((NPAD4, 128), jnp.float32),
)


def _dinv_g1_body(dp_ref, p_ref, dinv_ref, g_ref):
    dv = lax.rsqrt(dp_ref[0] + dp_ref[1] + 2.0)
    dinv_ref[...] = dv
    g_ref[...] = dv * p_ref[...]


_dinv_g1_tc = pl.pallas_call(
    _dinv_g1_body,
    grid=(GRID4,),
    in_specs=[
        pl.BlockSpec((2, RB4, 128), lambda i: (0, i, 0)),
        pl.BlockSpec((RB4, 128), lambda i: (i, 0)),
    ],
    out_specs=[
        pl.BlockSpec((RB4, 128), lambda i: (i, 0)),
        pl.BlockSpec((RB4, 128), lambda i: (i, 0)),
    ],
    out_shape=[
        jax.ShapeDtypeStruct((NPAD4, 128), jnp.float32),
        jax.ShapeDtypeStruct((NPAD4, 128), jnp.float32),
    ],
)


def _mid_body(s_ref, g_ref, dinv_ref, w_ref, b_ref, o_ref):
    i = pl.program_id(0)
    dv = dinv_ref[...]
    h = jnp.maximum(dv * (s_ref[0] + s_ref[1] + 2.0 * g_ref[...])
                    + b_ref[...], 0.0)
    p = jnp.dot(h, w_ref[...], preferred_element_type=jnp.float32)
    rows = i * RB4 + lax.broadcasted_iota(jnp.int32, (RB4, 1), 0)
    o_ref[...] = jnp.where(rows < N4, dv * p, 0.0)


_mid_tc = pl.pallas_call(
    _mid_body,
    grid=(GRID4,),
    in_specs=[
        pl.BlockSpec((2, RB4, 128), lambda i: (0, i, 0)),
        pl.BlockSpec((RB4, 128), lambda i: (i, 0)),
        pl.BlockSpec((RB4, 128), lambda i: (i, 0)),
        pl.BlockSpec((128, 128), lambda i: (0, 0)),
        pl.BlockSpec((1, 128), lambda i: (0, 0)),
    ],
    out_specs=pl.BlockSpec((RB4, 128), lambda i: (i, 0)),
    out_shape=jax.ShapeDtypeStruct((NPAD4, 128), jnp.float32),
)


def _final_body(s_ref, g_ref, dinv_ref, b_ref, bt_ref, wo_ref, bo_ref,
                o_ref, acc_ref):
    i = pl.program_id(0)

    @pl.when(i == 0)
    def _init():
        acc_ref[...] = jnp.zeros_like(acc_ref)

    h = jnp.maximum(
        dinv_ref[...] * (s_ref[0] + s_ref[1] + 2.0 * g_ref[...]) + b_ref[...],
        0.0)
    for q in range(4):
        bq = bt_ref[q:q + 1, :]
        oh = (lax.broadcasted_iota(jnp.int32, (G, RB4), 0)
              == jnp.broadcast_to(bq, (G, RB4))).astype(jnp.float32)
        acc_ref[...] += jnp.dot(oh, h[:, 32 * q:32 * (q + 1)],
                                preferred_element_type=jnp.float32)

    @pl.when(i == pl.num_programs(0) - 1)
    def _emit():
        o_ref[...] = (jnp.dot(acc_ref[...], wo_ref[...],
                              preferred_element_type=jnp.float32)
                      + bo_ref[...])


_final_tc = pl.pallas_call(
    _final_body,
    grid=(GRID4,),
    in_specs=[
        pl.BlockSpec((2, RB4, 128), lambda i: (0, i, 0)),
        pl.BlockSpec((RB4, 128), lambda i: (i, 0)),
        pl.BlockSpec((RB4, 128), lambda i: (i, 0)),
        pl.BlockSpec((1, 128), lambda i: (0, 0)),
        pl.BlockSpec((4, RB4), lambda i: (0, i)),
        pl.BlockSpec((DIM, 1), lambda i: (0, 0)),
        pl.BlockSpec((1, 1), lambda i: (0, 0)),
    ],
    out_specs=pl.BlockSpec((G, 1), lambda i: (0, 0)),
    out_shape=jax.ShapeDtypeStruct((G, 1), jnp.float32),
    scratch_shapes=[pltpu.VMEM((G, DIM), jnp.float32)],
)


def kernel(x, edge_index, batch, W1, b1, W2, b2, Wo, bo):
    src = edge_index[0]
    dst = edge_index[1]
    npad_rows = NPAD - N
    pad_idx = N + (jnp.arange(EPAD - E, dtype=jnp.int32) % npad_rows)
    srcp = jnp.concatenate([src, pad_idx]).reshape(NW, CHUNKS, CW)
    dstp = jnp.concatenate([dst, pad_idx]).reshape(NW, CHUNKS, CW)
    extra = N + (jnp.arange(NW * 2 * KG * CW, dtype=jnp.int32)
                 % npad_rows).reshape(NW, 2 * KG, CW)
    srcpa = jnp.concatenate([srcp, extra], axis=1)
    dstpa = jnp.concatenate([dstp, extra], axis=1)
    x4 = jnp.pad(x, ((0, npad_rows), (0, 0))).reshape(NPAD4, 4 * DF)
    batch4 = jnp.concatenate(
        [batch, jnp.full((npad_rows,), G, jnp.int32)]).reshape(NPAD4, 4).T
    eye4 = jnp.eye(4, dtype=jnp.float32)
    w1bd = jnp.kron(eye4, W1)
    w2bd = jnp.kron(eye4, W2)
    b1t = jnp.tile(b1, 4).reshape(1, 128)
    b2t = jnp.tile(b2, 4).reshape(1, 128)
    bor = bo.reshape(1, 1)

    _deg_sc, _edge_sc = _sc_kernels()
    deg_parts = _deg_sc(dstp)
    degs = deg_parts.reshape(NC, NPAD4, 128)
    p1 = _pre_tc(x4, w1bd)
    dinvp, g1p = _dinv_g1_tc(degs, p1)

    s1 = _edge_sc(g1p.reshape(NPAD, DIM), srcpa, dstpa)
    g2p = _mid_tc(s1.reshape(NC, NPAD4, 128), g1p, dinvp, w2bd, b1t)
    s2 = _edge_sc(g2p.reshape(NPAD, DIM), srcpa, dstpa)
    g3p = _mid_tc(s2.reshape(NC, NPAD4, 128), g2p, dinvp, w2bd, b2t)
    s3 = _edge_sc(g3p.reshape(NPAD, DIM), srcpa, dstpa)
    return _final_tc(s3.reshape(NC, NPAD4, 128), g3p, dinvp, b2t, batch4,
                     Wo, bor)


# speedup vs baseline: 1.1652x; 1.0139x over previous
import functools

import jax
import jax.numpy as jnp
from jax import lax
from jax.experimental import pallas as pl
from jax.experimental.pallas import tpu as pltpu
from jax.experimental.pallas import tpu_sc as plsc

N = 10000
E = 320000
DF = 128
DIM = 32
G = 64

NC = 2
NS = 16
NW = NC * NS
NPAD = 10240
EPAD = 327680
EPW = EPAD // NW
CW = 128
CHUNKS = EPW // CW
KG = 2
CHUNKS_A = CHUNKS + 2 * KG
STRIPE = NPAD // NS

RB = 1280
GRID = NPAD // RB

DEGW = DIM


def _deg_sc_body(dst_hbm, out_hbm, dst_v, ones_v, stage_v, acc_sh, sem):
    c = lax.axis_index("c")
    s = lax.axis_index("s")
    wid = s * NC + c
    pltpu.sync_copy(dst_hbm.at[wid], dst_v)

    one16 = jnp.ones((16,), jnp.float32)
    zero16 = jnp.zeros((16,), jnp.float32)

    def fill_body(i, carry):
        ones_v[i, pl.ds(0, 16)] = one16
        ones_v[i, pl.ds(16, 16)] = one16
        return carry

    lax.fori_loop(0, CW, fill_body, 0, unroll=8)

    def zero_body(i, carry):
        stage_v[i, pl.ds(0, 16)] = zero16
        stage_v[i, pl.ds(16, 16)] = zero16
        return carry

    lax.fori_loop(0, STRIPE, zero_body, 0, unroll=8)
    pltpu.sync_copy(stage_v, acc_sh.at[pl.ds(s * STRIPE, STRIPE)])
    plsc.subcore_barrier()

    def body(j, carry):
        pltpu.async_copy(ones_v, acc_sh.at[dst_v.at[j]], sem, add=True)
        return carry

    lax.fori_loop(0, CHUNKS, body, 0)

    def drain(j, carry):
        pltpu.make_async_copy(ones_v, acc_sh.at[dst_v.at[0]], sem).wait()
        return carry

    lax.fori_loop(0, CHUNKS, drain, 0)
    plsc.subcore_barrier()

    pltpu.sync_copy(acc_sh.at[pl.ds(s * STRIPE, STRIPE)], stage_v)
    pltpu.sync_copy(stage_v, out_hbm.at[c, pl.ds(s * STRIPE, STRIPE)])


def _edge_sc_body(g_hbm, src_hbm, dst_hbm, out_hbm,
                  src_v, dst_v, rows_a, rows_b, stage_v, g_sh, acc_sh,
                  gsem_a, gsem_b, ssem_a, ssem_b):
    c = lax.axis_index("c")
    s = lax.axis_index("s")
    wid = s * NC + c
    pltpu.sync_copy(src_hbm.at[wid], src_v)
    pltpu.sync_copy(dst_hbm.at[wid], dst_v)

    pltpu.sync_copy(g_hbm.at[pl.ds(s * STRIPE, STRIPE)], stage_v)
    pltpu.sync_copy(stage_v, g_sh.at[pl.ds(s * STRIPE, STRIPE)])

    zero16 = jnp.zeros((16,), jnp.float32)

    def zero_body(i, carry):
        stage_v[i, pl.ds(0, 16)] = zero16
        stage_v[i, pl.ds(16, 16)] = zero16
        return carry

    lax.fori_loop(0, STRIPE, zero_body, 0, unroll=8)
    pltpu.sync_copy(stage_v, acc_sh.at[pl.ds(s * STRIPE, STRIPE)])
    plsc.subcore_barrier()

    def fire_gathers(grp, buf, gsem):
        for b in range(KG):
            pltpu.async_copy(g_sh.at[src_v.at[grp * KG + b]], buf.at[b], gsem)

    def wait_gathers(buf, gsem):
        for b in range(KG):
            pltpu.make_async_copy(g_sh.at[src_v.at[0]], buf.at[b], gsem).wait()

    def fire_scatters(grp, buf, ssem):
        for b in range(KG):
            pltpu.async_copy(buf.at[b], acc_sh.at[dst_v.at[grp * KG + b]],
                             ssem, add=True)

    def wait_scatters(buf, ssem):
        for b in range(KG):
            pltpu.make_async_copy(buf.at[b], acc_sh.at[dst_v.at[0]],
                                  ssem).wait()

    fire_gathers(0, rows_a, gsem_a)
    fire_gathers(1, rows_b, gsem_b)

    def body(i, carry):
        ga = 2 * i
        wait_gathers(rows_a, gsem_a)
        fire_scatters(ga, rows_a, ssem_a)
        wait_gathers(rows_b, gsem_b)
        fire_scatters(ga + 1, rows_b, ssem_b)
        wait_scatters(rows_a, ssem_a)
        fire_gathers(ga + 2, rows_a, gsem_a)
        wait_scatters(rows_b, ssem_b)
        fire_gathers(ga + 3, rows_b, gsem_b)
        return carry

    lax.fori_loop(0, CHUNKS // (2 * KG), body, 0)
    wait_gathers(rows_a, gsem_a)
    wait_gathers(rows_b, gsem_b)
    plsc.subcore_barrier()

    pltpu.sync_copy(acc_sh.at[pl.ds(s * STRIPE, STRIPE)], stage_v)
    pltpu.sync_copy(stage_v, out_hbm.at[c, pl.ds(s * STRIPE, STRIPE)])


@functools.lru_cache(maxsize=1)
def _sc_kernels():
    mesh = plsc.VectorSubcoreMesh(
        core_axis_name="c", subcore_axis_name="s",
        num_cores=NC, num_subcores=NS)
    deg = pl.kernel(
        _deg_sc_body,
        out_type=jax.ShapeDtypeStruct((NC, NPAD, DEGW), jnp.float32),
        mesh=mesh,
        compiler_params=pltpu.CompilerParams(use_tc_tiling_on_sc=False),
        scratch_types=[
            pltpu.VMEM((CHUNKS, CW), jnp.int32),
            pltpu.VMEM((CW, DEGW), jnp.float32),
            pltpu.VMEM((STRIPE, DEGW), jnp.float32),
            pltpu.VMEM_SHARED((NPAD, DEGW), jnp.float32),
            pltpu.SemaphoreType.DMA,
        ],
    )
    edge = pl.kernel(
        _edge_sc_body,
        out_type=jax.ShapeDtypeStruct((NC, NPAD, DIM), jnp.float32),
        mesh=mesh,
        compiler_params=pltpu.CompilerParams(use_tc_tiling_on_sc=False),
        scratch_types=[
            pltpu.VMEM((CHUNKS_A, CW), jnp.int32),
            pltpu.VMEM((CHUNKS_A, CW), jnp.int32),
            pltpu.VMEM((KG, CW, DIM), jnp.float32),
            pltpu.VMEM((KG, CW, DIM), jnp.float32),
            pltpu.VMEM((STRIPE, DIM), jnp.float32),
            pltpu.VMEM_SHARED((NPAD, DIM), jnp.float32),
            pltpu.VMEM_SHARED((NPAD, DIM), jnp.float32),
            pltpu.SemaphoreType.DMA,
            pltpu.SemaphoreType.DMA,
            pltpu.SemaphoreType.DMA,
            pltpu.SemaphoreType.DMA,
        ],
    )
    return deg, edge


NPAD4 = NPAD // 4
N4 = N // 4
RB4 = 640
GRID4 = NPAD4 // RB4


def _pre_body(x_ref, w_ref, o_ref):
    o_ref[...] = jnp.dot(x_ref[...], w_ref[...],
                         preferred_element_type=jnp.float32)


_pre_tc = pl.pallas_call(
    _pre_body,
    grid=(GRID4,),
    in_specs=[
        pl.BlockSpec((RB4, 4 * DF), lambda i: (i, 0)),
        pl.BlockSpec((4 * DF, 128), lambda i: (0, 0)),
    ],
    out_specs=pl.BlockSpec((RB4, 128), lambda i: (i, 0)),
    out_shape=jax.ShapeDtypeStruct((NPAD4, 128), jnp.float32),
)


def _dinv_g1_body(dp_ref, p_ref, dinv_ref, g_ref):
    dv = lax.rsqrt(dp_ref[0] + dp_ref[1] + 2.0)
    dinv_ref[...] = dv
    g_ref[...] = dv * p_ref[...]


_dinv_g1_tc = pl.pallas_call(
    _dinv_g1_body,
    grid=(GRID4,),
    in_specs=[
        pl.BlockSpec((2, RB4, 128), lambda i: (0, i, 0)),
        pl.BlockSpec((RB4, 128), lambda i: (i, 0)),
    ],
    out_specs=[
        pl.BlockSpec((RB4, 128), lambda i: (i, 0)),
        pl.BlockSpec((RB4, 128), lambda i: (i, 0)),
    ],
    out_shape=[
        jax.ShapeDtypeStruct((NPAD4, 128), jnp.float32),
        jax.ShapeDtypeStruct((NPAD4, 128), jnp.float32),
    ],
)


def _mid_body(s_ref, g_ref, dinv_ref, w_ref, b_ref, o_ref):
    i = pl.program_id(0)
    dv = dinv_ref[...]
    h = jnp.maximum(dv * (s_ref[0] + s_ref[1] + 2.0 * g_ref[...])
                    + b_ref[...], 0.0)
    p = jnp.dot(h, w_ref[...], preferred_element_type=jnp.float32)
    rows = i * RB4 + lax.broadcasted_iota(jnp.int32, (RB4, 1), 0)
    o_ref[...] = jnp.where(rows < N4, dv * p, 0.0)


_mid_tc = pl.pallas_call(
    _mid_body,
    grid=(GRID4,),
    in_specs=[
        pl.BlockSpec((2, RB4, 128), lambda i: (0, i, 0)),
        pl.BlockSpec((RB4, 128), lambda i: (i, 0)),
        pl.BlockSpec((RB4, 128), lambda i: (i, 0)),
        pl.BlockSpec((128, 128), lambda i: (0, 0)),
        pl.BlockSpec((1, 128), lambda i: (0, 0)),
    ],
    out_specs=pl.BlockSpec((RB4, 128), lambda i: (i, 0)),
    out_shape=jax.ShapeDtypeStruct((NPAD4, 128), jnp.float32),
)


def _final_body(s_ref, g_ref, dinv_ref, b_ref, bt_ref, wo_ref, bo_ref,
                o_ref, acc_ref):
    i = pl.program_id(0)

    @pl.when(i == 0)
    def _init():
        acc_ref[...] = jnp.zeros_like(acc_ref)

    h = jnp.maximum(
        dinv_ref[...] * (s_ref[0] + s_ref[1] + 2.0 * g_ref[...]) + b_ref[...],
        0.0)
    for q in range(4):
        bq = bt_ref[q:q + 1, :]
        oh = (lax.broadcasted_iota(jnp.int32, (G, RB4), 0)
              == jnp.broadcast_to(bq, (G, RB4))).astype(jnp.float32)
        acc_ref[...] += jnp.dot(oh, h[:, 32 * q:32 * (q + 1)],
                                preferred_element_type=jnp.float32)

    @pl.when(i == pl.num_programs(0) - 1)
    def _emit():
        o_ref[...] = (jnp.dot(acc_ref[...], wo_ref[...],
                              preferred_element_type=jnp.float32)
                      + bo_ref[...])


_final_tc = pl.pallas_call(
    _final_body,
    grid=(GRID4,),
    in_specs=[
        pl.BlockSpec((2, RB4, 128), lambda i: (0, i, 0)),
        pl.BlockSpec((RB4, 128), lambda i: (i, 0)),
        pl.BlockSpec((RB4, 128), lambda i: (i, 0)),
        pl.BlockSpec((1, 128), lambda i: (0, 0)),
        pl.BlockSpec((4, RB4), lambda i: (0, i)),
        pl.BlockSpec((DIM, 1), lambda i: (0, 0)),
        pl.BlockSpec((1, 1), lambda i: (0, 0)),
    ],
    out_specs=pl.BlockSpec((G, 1), lambda i: (0, 0)),
    out_shape=jax.ShapeDtypeStruct((G, 1), jnp.float32),
    scratch_shapes=[pltpu.VMEM((G, DIM), jnp.float32)],
)


def kernel(x, edge_index, batch, W1, b1, W2, b2, Wo, bo):
    src = edge_index[0]
    dst = edge_index[1]
    npad_rows = NPAD - N
    pad_idx = N + (jnp.arange(EPAD - E, dtype=jnp.int32) % npad_rows)
    srcp = jnp.concatenate([src, pad_idx]).reshape(NW, CHUNKS, CW)
    dstp = jnp.concatenate([dst, pad_idx]).reshape(NW, CHUNKS, CW)
    extra = N + (jnp.arange(NW * 2 * KG * CW, dtype=jnp.int32)
                 % npad_rows).reshape(NW, 2 * KG, CW)
    srcpa = jnp.concatenate([srcp, extra], axis=1)
    dstpa = jnp.concatenate([dstp, extra], axis=1)
    x4 = jnp.pad(x, ((0, npad_rows), (0, 0))).reshape(NPAD4, 4 * DF)
    batch4 = jnp.concatenate(
        [batch, jnp.full((npad_rows,), G, jnp.int32)]).reshape(NPAD4, 4).T
    eye4 = jnp.eye(4, dtype=jnp.float32)
    w1bd = jnp.kron(eye4, W1)
    w2bd = jnp.kron(eye4, W2)
    b1t = jnp.tile(b1, 4).reshape(1, 128)
    b2t = jnp.tile(b2, 4).reshape(1, 128)
    bor = bo.reshape(1, 1)

    _deg_sc, _edge_sc = _sc_kernels()
    deg_parts = _deg_sc(dstp)
    degs = deg_parts.reshape(NC, NPAD4, 128)
    p1 = _pre_tc(x4, w1bd)
    dinvp, g1p = _dinv_g1_tc(degs, p1)

    s1 = _edge_sc(g1p.reshape(NPAD, DIM), srcpa, dstpa)
    g2p = _mid_tc(s1.reshape(NC, NPAD4, 128), g1p, dinvp, w2bd, b1t)
    s2 = _edge_sc(g2p.reshape(NPAD, DIM), srcpa, dstpa)
    g3p = _mid_tc(s2.reshape(NC, NPAD4, 128), g2p, dinvp, w2bd, b2t)
    s3 = _edge_sc(g3p.reshape(NPAD, DIM), srcpa, dstpa)
    return _final_tc(s3.reshape(NC, NPAD4, 128), g3p, dinvp, b2t, batch4,
                     Wo, bor)
